# baseline (speedup 1.0000x reference)
import functools

import jax
import jax.numpy as jnp
from jax import lax
from jax.experimental import pallas as pl
from jax.experimental.pallas import tpu as pltpu
from jax.experimental.pallas import tpu_sc as plsc

_N = 50000
_E = 800000
_H = 64
_EPS = 1e-5

_BM = 1000
_GRID = _N // _BM

_NC = 2
_NS = 16
_LW = 128
_EP = 802816
_ER = _EP // _LW
_RPT = _ER // _NS
_CH = 8
_NOUT = _RPT // _CH
_CHD = 8
_NCHD = _ER // _CHD
_NA = 50048
_ZPT = _NA // _NS
_ZC = 184
_OC = 400
_NOC = _N // _OC
_QW = 16

_SC_MESH = plsc.VectorSubcoreMesh(core_axis_name="c", subcore_axis_name="s")
_SC_PARAMS = pltpu.CompilerParams(use_tc_tiling_on_sc=False)


def _zero_stage(stage_ref, nrows):
  zero = jnp.zeros((16,), jnp.float32)

  def body(i, carry):
    stage_ref[i, pl.ds(0, 16)] = zero
    return carry

  lax.fori_loop(0, nrows, body, 0)


def _zero_acc(stage_ref, acc_ref, s):
  def zcp(j, carry):
    pltpu.sync_copy(stage_ref.at[pl.ds(0, _ZC)],
                    acc_ref.at[pl.ds(s * _ZPT + j * _ZC, _ZC)])
    return carry

  lax.fori_loop(0, _ZPT // _ZC, zcp, 0)


def _copy_out(stage_ref, acc_ref, out_hbm, c, s, base=0):
  def ocp(jj, carry):
    r0 = (jj * _NS + s) * _OC
    pltpu.sync_copy(acc_ref.at[pl.ds(r0, _OC)], stage_ref)
    pltpu.sync_copy(stage_ref, out_hbm.at[pl.ds(base + c * _N + r0, _OC)])
    return carry

  nchunks = jnp.where(s < _NOC - (_NOC // _NS) * _NS,
                      _NOC // _NS + 1, _NOC // _NS)
  lax.fori_loop(0, nchunks, ocp, 0)


@functools.partial(
    pl.kernel,
    out_type=jax.ShapeDtypeStruct((2 * _N, _QW), jnp.float32),
    mesh=_SC_MESH,
    scratch_types=[
        pltpu.VMEM((_CHD, _LW), jnp.int32),
        pltpu.VMEM((_LW, _QW), jnp.float32),
        pltpu.VMEM((_OC, _QW), jnp.float32),
        pltpu.VMEM_SHARED((_NA, _QW), jnp.float32),
        pltpu.SemaphoreType.DMA,
    ],
    compiler_params=_SC_PARAMS,
)
def _sc_deg(dst_hbm, out_hbm, idx_v, ones_v, stage_v, acc_sh, sem):
  c = lax.axis_index("c")
  s = lax.axis_index("s")
  w = s * _NC + c
  one = jnp.ones((16,), jnp.float32)

  def fill(i, carry):
    ones_v[i, :] = one
    return carry

  lax.fori_loop(0, _LW, fill, 0)
  _zero_stage(stage_v, _OC)
  _zero_acc(stage_v, acc_sh, s)
  plsc.subcore_barrier()

  def chunk(jj, carry):
    pltpu.sync_copy(dst_hbm.at[pl.ds((jj * _NS * _NC + w) * _CHD, _CHD)],
                    idx_v)
    copies = [pltpu.async_copy(ones_v, acc_sh.at[idx_v.at[r]], sem, add=True)
              for r in range(_CHD)]
    for d in copies:
      d.wait()
    return carry

  nw = _NS * _NC
  nchunks = jnp.where(w < _NCHD - (_NCHD // nw) * nw,
                      _NCHD // nw + 1, _NCHD // nw)
  lax.fori_loop(0, nchunks, chunk, 0)
  plsc.subcore_barrier()
  _copy_out(stage_v, acc_sh, out_hbm, c, s)


@functools.partial(
    pl.kernel,
    out_type=jax.ShapeDtypeStruct((4 * _N, _QW), jnp.float32),
    mesh=_SC_MESH,
    scratch_types=[
        pltpu.VMEM((_CH, _LW), jnp.int32),
        pltpu.VMEM((_CH, _LW), jnp.int32),
        pltpu.VMEM((_CH, _LW), jnp.int32),
        pltpu.VMEM((_CH, _LW), jnp.int32),
        pltpu.VMEM((_CH * _LW, _QW), jnp.float32),
        pltpu.VMEM((_CH * _LW, _QW), jnp.float32),
        pltpu.VMEM((_OC, _QW), jnp.float32),
        pltpu.VMEM_SHARED((_NA, _QW), jnp.float32),
        pltpu.SemaphoreType.DMA,
        pltpu.SemaphoreType.DMA,
        pltpu.SemaphoreType.DMA,
        pltpu.SemaphoreType.DMA,
        pltpu.SemaphoreType.DMA,
        pltpu.SemaphoreType.DMA,
        pltpu.SemaphoreType.DMA,
        pltpu.SemaphoreType.DMA,
    ],
    compiler_params=_SC_PARAMS,
)
def _sc_agg(g_hbm, src_hbm, dst_hbm, out_hbm,
            src_v0, src_v1, dst_v0, dst_v1, rows_v0, rows_v1, stage_v,
            acc_sh, semi_s0, semi_s1, semi_d0, semi_d1, semg0, semg1,
            sems0, sems1):
  c = lax.axis_index("c")
  s = lax.axis_index("s")
  src_v = (src_v0, src_v1)
  dst_v = (dst_v0, dst_v1)
  rows_v = (rows_v0, rows_v1)
  semi_s = (semi_s0, semi_s1)
  semi_d = (semi_d0, semi_d1)
  semg = (semg0, semg1)
  sems = (sems0, sems1)

  def src_fire(pp, k, p):
    pltpu.async_copy(src_hbm.at[pp, c, pl.ds(s * _RPT + k * _CH, _CH)],
                     src_v[p], semi_s[p])

  def src_wait(p):
    pltpu.make_async_copy(src_hbm.at[0, c, pl.ds(0, _CH)], src_v[p],
                          semi_s[p]).wait()

  def dst_fire(k, p):
    pltpu.async_copy(dst_hbm.at[pl.ds(s * _RPT + k * _CH, _CH)],
                     dst_v[p], semi_d[p])

  def dst_wait(p):
    pltpu.make_async_copy(dst_hbm.at[pl.ds(0, _CH)], dst_v[p],
                          semi_d[p]).wait()

  def gather_fire(p):
    for r in range(_CH):
      pltpu.async_copy(g_hbm.at[src_v[p].at[r]],
                       rows_v[p].at[pl.ds(r * _LW, _LW)], semg[p])

  def gather_wait(p):
    for r in range(_CH):
      pltpu.make_async_copy(g_hbm.at[src_v[p].at[r]],
                            rows_v[p].at[pl.ds(r * _LW, _LW)],
                            semg[p]).wait()

  def scatter_fire(p):
    for r in range(_CH):
      pltpu.async_copy(rows_v[p].at[pl.ds(r * _LW, _LW)],
                       acc_sh.at[dst_v[p].at[r]], sems[p], add=True)

  def scatter_wait(p):
    for r in range(_CH):
      pltpu.make_async_copy(rows_v[p].at[pl.ds(r * _LW, _LW)],
                            acc_sh.at[dst_v[p].at[r]], sems[p]).wait()

  for pp in range(2):
    _zero_stage(stage_v, _OC)
    _zero_acc(stage_v, acc_sh, s)
    plsc.subcore_barrier()

    src_fire(pp, 0, 0)
    src_wait(0)
    dst_fire(0, 0)
    gather_fire(0)
    src_fire(pp, 1, 1)
    src_wait(1)
    dst_fire(1, 1)
    gather_fire(1)
    gather_wait(0)
    dst_wait(0)
    scatter_fire(0)
    src_fire(pp, 2, 0)

    def steady(k, p):
      q = 1 - p
      src_wait(p)
      scatter_wait(p)
      dst_fire(k, p)
      gather_fire(p)
      gather_wait(q)
      dst_wait(q)
      scatter_fire(q)
      src_fire(pp, k + 1, q)

    def pair(jj, carry):
      k = 2 + 2 * jj
      steady(k, 0)
      steady(k + 1, 1)
      return carry

    lax.fori_loop(0, (_NOUT - 3) // 2, pair, 0)

    k_last = _NOUT - 1
    src_wait(0)
    scatter_wait(0)
    dst_fire(k_last, 0)
    gather_fire(0)
    gather_wait(1)
    dst_wait(1)
    scatter_fire(1)
    gather_wait(0)
    dst_wait(0)
    scatter_fire(0)
    scatter_wait(1)
    scatter_wait(0)

    plsc.subcore_barrier()
    _copy_out(stage_v, acc_sh, out_hbm, c, s, base=2 * pp * _N)
    if pp == 0:
      plsc.subcore_barrier()


def _split_q(g_ref, g):
  for q in range(4):
    g_ref[q, :, :] = g[:, q * _QW:(q + 1) * _QW]


def _enc_body(x_ref, degp_ref, wenc_ref, benc_ref, lnw_ref, lnb_ref, w1t_ref,
              h0_ref, dinv_ref, g_ref):
  xb = x_ref[...]
  h = jnp.dot(xb, wenc_ref[...], preferred_element_type=jnp.float32)
  h = jnp.maximum(h + benc_ref[0:1, :], 0.0)
  m = jnp.mean(h, axis=1, keepdims=True)
  v = jnp.mean((h - m) ** 2, axis=1, keepdims=True)
  h0 = (h - m) * lax.rsqrt(v + _EPS) * lnw_ref[0:1, :] + lnb_ref[0:1, :]
  h0_ref[...] = h0
  deg = degp_ref[0, :, 0:1] + degp_ref[1, :, 0:1] + 1.0
  dinv = lax.rsqrt(deg)
  dinv_ref[...] = jnp.broadcast_to(dinv, (_BM, 8))
  g = dinv * jnp.dot(h0, w1t_ref[...], preferred_element_type=jnp.float32)
  _split_q(g_ref, g)


def _stat_body(agg_ref, g_ref, dinv_ref, b_ref, t_ref, st_ref):
  i = pl.program_id(0)
  a = jnp.concatenate([agg_ref[q] for q in range(4)], axis=1)
  gg = jnp.concatenate([g_ref[q] for q in range(4)], axis=1)
  dinv = dinv_ref[:, 0:1]
  t = jnp.maximum(dinv * (a + gg) + b_ref[0:1, :], 0.0)
  t_ref[...] = t
  s1 = jnp.sum(t, axis=0, keepdims=True)
  s2 = jnp.sum(t * t, axis=0, keepdims=True)
  p = jnp.concatenate([s1, s2, jnp.zeros((6, _H), jnp.float32)], axis=0)

  @pl.when(i == 0)
  def _():
    st_ref[...] = p

  @pl.when(i > 0)
  def _():
    st_ref[...] = st_ref[...] + p


def _norm_block(t_ref, st_ref, bnw_ref, bnb_ref, id_ref):
  st = st_ref[...]
  m = st[0:1, :] * (1.0 / _N)
  ex2 = st[1:2, :] * (1.0 / _N)
  var = ex2 - m * m
  bn = (t_ref[...] - m) * lax.rsqrt(var + _EPS) * bnw_ref[0:1, :] \
      + bnb_ref[0:1, :]
  u = bn + id_ref[...]
  rm = jnp.mean(u, axis=1, keepdims=True)
  rv = jnp.mean((u - rm) ** 2, axis=1, keepdims=True)
  return (u - rm) * lax.rsqrt(rv + _EPS)


def _apply_body(t_ref, st_ref, bnw_ref, bnb_ref, id_ref, dinv_ref, wt_ref,
                h_ref, g_ref):
  hn = _norm_block(t_ref, st_ref, bnw_ref, bnb_ref, id_ref)
  h_ref[...] = hn
  g = dinv_ref[:, 0:1] * jnp.dot(hn, wt_ref[...],
                                 preferred_element_type=jnp.float32)
  _split_q(g_ref, g)


def _final_body(t_ref, st_ref, bnw_ref, bnb_ref, id_ref,
                fc1t_ref, fc1b_ref, fc2t_ref, fc2b_ref, o_ref):
  hn = _norm_block(t_ref, st_ref, bnw_ref, bnb_ref, id_ref)
  y = jnp.dot(hn, fc1t_ref[...], preferred_element_type=jnp.float32)
  y = jnp.maximum(y + fc1b_ref[0:1, :], 0.0)
  z = jnp.dot(y, fc2t_ref[...], preferred_element_type=jnp.float32)
  o_ref[...] = jnp.tanh(z + fc2b_ref[0:1, :])


def _rows(i):
  return (i, 0)


def _fixed(*_):
  return (0, 0)


_B_X = pl.BlockSpec((_BM, 8), _rows)
_B_H = pl.BlockSpec((_BM, _H), _rows)
_B_D8 = pl.BlockSpec((_BM, 8), _rows)
_B_G = pl.BlockSpec((4, _BM, _QW), lambda i: (0, i, 0))
_B_DEGP = pl.BlockSpec((2, _BM, _QW), lambda i: (0, i, 0))
_B_P8 = pl.BlockSpec((8, _H), _fixed)
_B_P832 = pl.BlockSpec((8, 32), _fixed)
_B_P88 = pl.BlockSpec((8, 8), _fixed)
_B_W = pl.BlockSpec((_H, _H), _fixed)
_B_W832 = pl.BlockSpec((8, _H), _fixed)
_B_FC1 = pl.BlockSpec((_H, 32), _fixed)
_B_FC2 = pl.BlockSpec((32, 8), _fixed)
_B_O = pl.BlockSpec((_BM, 8), _rows)

_enc = pl.pallas_call(
    _enc_body,
    grid=(_GRID,),
    in_specs=[_B_X, _B_DEGP, _B_W832, _B_P8, _B_P8, _B_P8, _B_W],
    out_specs=[_B_H, _B_D8, _B_G],
    out_shape=[
        jax.ShapeDtypeStruct((_N, _H), jnp.float32),
        jax.ShapeDtypeStruct((_N, 8), jnp.float32),
        jax.ShapeDtypeStruct((4, _N, _QW), jnp.float32),
    ],
)

_stat = pl.pallas_call(
    _stat_body,
    grid=(_GRID,),
    in_specs=[_B_G, _B_G, _B_D8, _B_P8],
    out_specs=[_B_H, _B_P8],
    out_shape=[
        jax.ShapeDtypeStruct((_N, _H), jnp.float32),
        jax.ShapeDtypeStruct((8, _H), jnp.float32),
    ],
)

_apply = pl.pallas_call(
    _apply_body,
    grid=(_GRID,),
    in_specs=[_B_H, _B_P8, _B_P8, _B_P8, _B_H, _B_D8, _B_W],
    out_specs=[_B_H, _B_G],
    out_shape=[
        jax.ShapeDtypeStruct((_N, _H), jnp.float32),
        jax.ShapeDtypeStruct((4, _N, _QW), jnp.float32),
    ],
)

_final = pl.pallas_call(
    _final_body,
    grid=(_GRID,),
    in_specs=[_B_H, _B_P8, _B_P8, _B_P8, _B_H, _B_FC1, _B_P832, _B_FC2,
              _B_P88],
    out_specs=_B_O,
    out_shape=jax.ShapeDtypeStruct((_N, 8), jnp.float32),
)


def _row8(v):
  return jnp.broadcast_to(v.reshape(1, -1).astype(jnp.float32), (8, v.size))


def kernel(x, edge_index, W_enc, b_enc, ln_w, ln_b, bn_w, bn_b,
           W1, b1, W2, b2, W3, b3, fc1_w, fc1_b, fc2_w, fc2_b):
  src = edge_index[0].astype(jnp.int32)
  dst = edge_index[1].astype(jnp.int32)
  pad = _EP - _E
  srcp = jnp.concatenate([src, jnp.zeros((pad,), jnp.int32)])
  dstp = jnp.concatenate([dst, jnp.full((pad,), _N, jnp.int32)])
  srcq = jnp.stack([srcp, srcp + _N, srcp + 2 * _N, srcp + 3 * _N])
  srcq = srcq.reshape(2, 2, _ER, _LW)
  dst2 = dstp.reshape(_ER, _LW)

  xp = jnp.pad(x.astype(jnp.float32), ((0, 0), (0, 6)))
  wenc = jnp.pad(W_enc.astype(jnp.float32).T, ((0, 6), (0, 0)))
  w1t = W1.astype(jnp.float32).T
  w2t = W2.astype(jnp.float32).T
  w3t = W3.astype(jnp.float32).T
  fc1t = fc1_w.astype(jnp.float32).T
  fc2t = jnp.pad(fc2_w.astype(jnp.float32).T, ((0, 0), (0, 6)))
  fc2b = jnp.pad(fc2_b.astype(jnp.float32), (0, 6))

  degp = _sc_deg(dst2).reshape(2, _N, _QW)

  h0, dinv8, g = _enc(xp, degp, wenc, _row8(b_enc), _row8(ln_w),
                      _row8(ln_b), w1t)

  hid = h0
  for bvec, nxt in ((b1, w2t), (b2, w3t), (b3, None)):
    gt = g.reshape(4 * _N, _QW)
    agg = _sc_agg(gt, srcq, dst2).reshape(4, _N, _QW)
    t, st = _stat(agg, g, dinv8, _row8(bvec))
    if nxt is not None:
      hid, g = _apply(t, st, _row8(bn_w), _row8(bn_b), hid, dinv8, nxt)
    else:
      out8 = _final(t, st, _row8(bn_w), _row8(bn_b), hid, fc1t,
                    _row8(fc1_b), fc2t, _row8(fc2b))
  return out8[:, :2]


# speedup vs baseline: 1.0244x; 1.0244x over previous
import functools

import jax
import jax.numpy as jnp
from jax import lax
from jax.experimental import pallas as pl
from jax.experimental.pallas import tpu as pltpu
from jax.experimental.pallas import tpu_sc as plsc

_N = 50000
_E = 800000
_H = 64
_EPS = 1e-5

_BM = 1000
_GRID = _N // _BM

_NC = 2
_NS = 16
_LW = 128
_EP = 802816
_ER = _EP // _LW
_RPT = _ER // _NS
_CH = 8
_NOUT = _RPT // _CH
_CHD = 8
_NCHD = _ER // _CHD
_NA = 50048
_ZPT = _NA // _NS
_ZC = 184
_OC = 400
_NOC = _N // _OC
_QW = 16

_SC_MESH = plsc.VectorSubcoreMesh(core_axis_name="c", subcore_axis_name="s")
_SC_PARAMS = pltpu.CompilerParams(use_tc_tiling_on_sc=False)


def _zero_stage(stage_ref, nrows):
  zero = jnp.zeros((16,), jnp.float32)

  def body(i, carry):
    stage_ref[i, pl.ds(0, 16)] = zero
    return carry

  lax.fori_loop(0, nrows, body, 0)


def _zero_acc(stage_ref, acc_ref, s):
  def zcp(j, carry):
    pltpu.sync_copy(stage_ref.at[pl.ds(0, _ZC)],
                    acc_ref.at[pl.ds(s * _ZPT + j * _ZC, _ZC)])
    return carry

  lax.fori_loop(0, _ZPT // _ZC, zcp, 0)


def _copy_out(stage_ref, acc_ref, out_hbm, c, s, base=0):
  def ocp(jj, carry):
    r0 = (jj * _NS + s) * _OC
    pltpu.sync_copy(acc_ref.at[pl.ds(r0, _OC)], stage_ref)
    pltpu.sync_copy(stage_ref, out_hbm.at[pl.ds(base + c * _N + r0, _OC)])
    return carry

  nchunks = jnp.where(s < _NOC - (_NOC // _NS) * _NS,
                      _NOC // _NS + 1, _NOC // _NS)
  lax.fori_loop(0, nchunks, ocp, 0)


@functools.partial(
    pl.kernel,
    out_type=jax.ShapeDtypeStruct((2 * _N, _QW), jnp.float32),
    mesh=_SC_MESH,
    scratch_types=[
        pltpu.VMEM((_CHD, _LW), jnp.int32),
        pltpu.VMEM((_LW, _QW), jnp.float32),
        pltpu.VMEM((_OC, _QW), jnp.float32),
        pltpu.VMEM_SHARED((_NA, _QW), jnp.float32),
        pltpu.SemaphoreType.DMA,
    ],
    compiler_params=_SC_PARAMS,
)
def _sc_deg(dst_hbm, out_hbm, idx_v, ones_v, stage_v, acc_sh, sem):
  c = lax.axis_index("c")
  s = lax.axis_index("s")
  w = s * _NC + c
  one = jnp.ones((16,), jnp.float32)

  def fill(i, carry):
    ones_v[i, :] = one
    return carry

  lax.fori_loop(0, _LW, fill, 0)
  _zero_stage(stage_v, _OC)
  _zero_acc(stage_v, acc_sh, s)
  plsc.subcore_barrier()

  def chunk(jj, carry):
    pltpu.sync_copy(dst_hbm.at[pl.ds((jj * _NS * _NC + w) * _CHD, _CHD)],
                    idx_v)
    copies = [pltpu.async_copy(ones_v, acc_sh.at[idx_v.at[r]], sem, add=True)
              for r in range(_CHD)]
    for d in copies:
      d.wait()
    return carry

  nw = _NS * _NC
  nchunks = jnp.where(w < _NCHD - (_NCHD // nw) * nw,
                      _NCHD // nw + 1, _NCHD // nw)
  lax.fori_loop(0, nchunks, chunk, 0)
  plsc.subcore_barrier()
  _copy_out(stage_v, acc_sh, out_hbm, c, s)


@functools.partial(
    pl.kernel,
    out_type=jax.ShapeDtypeStruct((2 * _N, _QW), jnp.float32),
    mesh=_SC_MESH,
    scratch_types=[
        pltpu.VMEM((_CH, _LW), jnp.int32),
        pltpu.VMEM((_CH, _LW), jnp.int32),
        pltpu.VMEM((_CH, _LW), jnp.int32),
        pltpu.VMEM((_CH, _LW), jnp.int32),
        pltpu.VMEM((_CH * _LW, _QW), jnp.float32),
        pltpu.VMEM((_CH * _LW, _QW), jnp.float32),
        pltpu.VMEM((_OC, _QW), jnp.float32),
        pltpu.VMEM_SHARED((_NA, _QW), jnp.float32),
        pltpu.SemaphoreType.DMA,
        pltpu.SemaphoreType.DMA,
        pltpu.SemaphoreType.DMA,
        pltpu.SemaphoreType.DMA,
        pltpu.SemaphoreType.DMA,
        pltpu.SemaphoreType.DMA,
        pltpu.SemaphoreType.DMA,
        pltpu.SemaphoreType.DMA,
    ],
    compiler_params=_SC_PARAMS,
)
def _sc_agg(g_hbm, src_hbm, dst_hbm, out_hbm,
            src_v0, src_v1, dst_v0, dst_v1, rows_v0, rows_v1, stage_v,
            acc_sh, semi_s0, semi_s1, semi_d0, semi_d1, semg0, semg1,
            sems0, sems1):
  c = lax.axis_index("c")
  s = lax.axis_index("s")
  src_v = (src_v0, src_v1)
  dst_v = (dst_v0, dst_v1)
  rows_v = (rows_v0, rows_v1)
  semi_s = (semi_s0, semi_s1)
  semi_d = (semi_d0, semi_d1)
  semg = (semg0, semg1)
  sems = (sems0, sems1)

  def src_fire(k, p):
    pltpu.async_copy(src_hbm.at[c, pl.ds(s * _RPT + k * _CH, _CH)],
                     src_v[p], semi_s[p])

  def src_wait(p):
    pltpu.make_async_copy(src_hbm.at[c, pl.ds(0, _CH)], src_v[p],
                          semi_s[p]).wait()

  def dst_fire(k, p):
    pltpu.async_copy(dst_hbm.at[pl.ds(s * _RPT + k * _CH, _CH)],
                     dst_v[p], semi_d[p])

  def dst_wait(p):
    pltpu.make_async_copy(dst_hbm.at[pl.ds(0, _CH)], dst_v[p],
                          semi_d[p]).wait()

  def gather_fire(p):
    for r in range(_CH):
      pltpu.async_copy(g_hbm.at[src_v[p].at[r]],
                       rows_v[p].at[pl.ds(r * _LW, _LW)], semg[p])

  def gather_wait(p):
    for r in range(_CH):
      pltpu.make_async_copy(g_hbm.at[src_v[p].at[r]],
                            rows_v[p].at[pl.ds(r * _LW, _LW)],
                            semg[p]).wait()

  def scatter_fire(p):
    for r in range(_CH):
      pltpu.async_copy(rows_v[p].at[pl.ds(r * _LW, _LW)],
                       acc_sh.at[dst_v[p].at[r]], sems[p], add=True)

  def scatter_wait(p):
    for r in range(_CH):
      pltpu.make_async_copy(rows_v[p].at[pl.ds(r * _LW, _LW)],
                            acc_sh.at[dst_v[p].at[r]], sems[p]).wait()

  _zero_stage(stage_v, _OC)
  _zero_acc(stage_v, acc_sh, s)
  plsc.subcore_barrier()

  src_fire(0, 0)
  src_wait(0)
  dst_fire(0, 0)
  gather_fire(0)
  src_fire(1, 1)
  src_wait(1)
  dst_fire(1, 1)
  gather_fire(1)
  gather_wait(0)
  dst_wait(0)
  scatter_fire(0)
  src_fire(2, 0)

  def steady(k, p):
    q = 1 - p
    src_wait(p)
    scatter_wait(p)
    dst_fire(k, p)
    gather_fire(p)
    gather_wait(q)
    dst_wait(q)
    scatter_fire(q)
    src_fire(k + 1, q)

  def pair(jj, carry):
    k = 2 + 2 * jj
    steady(k, 0)
    steady(k + 1, 1)
    return carry

  lax.fori_loop(0, (_NOUT - 3) // 2, pair, 0)

  k_last = _NOUT - 1
  src_wait(0)
  scatter_wait(0)
  dst_fire(k_last, 0)
  gather_fire(0)
  gather_wait(1)
  dst_wait(1)
  scatter_fire(1)
  gather_wait(0)
  dst_wait(0)
  scatter_fire(0)
  scatter_wait(1)
  scatter_wait(0)

  plsc.subcore_barrier()
  _copy_out(stage_v, acc_sh, out_hbm, c, s)


def _split_q(g_ref, g):
  for q in range(4):
    g_ref[q, :, :] = g[:, q * _QW:(q + 1) * _QW]


def _enc_body(x_ref, degp_ref, wenc_ref, benc_ref, lnw_ref, lnb_ref, w1t_ref,
              h0_ref, dinv_ref, g_ref):
  xb = x_ref[...]
  h = jnp.dot(xb, wenc_ref[...], preferred_element_type=jnp.float32)
  h = jnp.maximum(h + benc_ref[0:1, :], 0.0)
  m = jnp.mean(h, axis=1, keepdims=True)
  v = jnp.mean((h - m) ** 2, axis=1, keepdims=True)
  h0 = (h - m) * lax.rsqrt(v + _EPS) * lnw_ref[0:1, :] + lnb_ref[0:1, :]
  h0_ref[...] = h0
  deg = degp_ref[0, :, 0:1] + degp_ref[1, :, 0:1] + 1.0
  dinv = lax.rsqrt(deg)
  dinv_ref[...] = jnp.broadcast_to(dinv, (_BM, 8))
  g = dinv * jnp.dot(h0, w1t_ref[...], preferred_element_type=jnp.float32)
  _split_q(g_ref, g)


def _stat_body(agga_ref, aggb_ref, g_ref, dinv_ref, b_ref, t_ref, st_ref):
  i = pl.program_id(0)
  a = jnp.concatenate([agga_ref[0], agga_ref[1], aggb_ref[0], aggb_ref[1]],
                      axis=1)
  gg = jnp.concatenate([g_ref[q] for q in range(4)], axis=1)
  dinv = dinv_ref[:, 0:1]
  t = jnp.maximum(dinv * (a + gg) + b_ref[0:1, :], 0.0)
  t_ref[...] = t
  s1 = jnp.sum(t, axis=0, keepdims=True)
  s2 = jnp.sum(t * t, axis=0, keepdims=True)
  p = jnp.concatenate([s1, s2, jnp.zeros((6, _H), jnp.float32)], axis=0)

  @pl.when(i == 0)
  def _():
    st_ref[...] = p

  @pl.when(i > 0)
  def _():
    st_ref[...] = st_ref[...] + p


def _norm_block(t_ref, st_ref, bnw_ref, bnb_ref, id_ref):
  st = st_ref[...]
  m = st[0:1, :] * (1.0 / _N)
  ex2 = st[1:2, :] * (1.0 / _N)
  var = ex2 - m * m
  bn = (t_ref[...] - m) * lax.rsqrt(var + _EPS) * bnw_ref[0:1, :] \
      + bnb_ref[0:1, :]
  u = bn + id_ref[...]
  rm = jnp.mean(u, axis=1, keepdims=True)
  rv = jnp.mean((u - rm) ** 2, axis=1, keepdims=True)
  return (u - rm) * lax.rsqrt(rv + _EPS)


def _apply_body(t_ref, st_ref, bnw_ref, bnb_ref, id_ref, dinv_ref, wt_ref,
                h_ref, g_ref):
  hn = _norm_block(t_ref, st_ref, bnw_ref, bnb_ref, id_ref)
  h_ref[...] = hn
  g = dinv_ref[:, 0:1] * jnp.dot(hn, wt_ref[...],
                                 preferred_element_type=jnp.float32)
  _split_q(g_ref, g)


def _final_body(t_ref, st_ref, bnw_ref, bnb_ref, id_ref,
                fc1t_ref, fc1b_ref, fc2t_ref, fc2b_ref, o_ref):
  hn = _norm_block(t_ref, st_ref, bnw_ref, bnb_ref, id_ref)
  y = jnp.dot(hn, fc1t_ref[...], preferred_element_type=jnp.float32)
  y = jnp.maximum(y + fc1b_ref[0:1, :], 0.0)
  z = jnp.dot(y, fc2t_ref[...], preferred_element_type=jnp.float32)
  o_ref[...] = jnp.tanh(z + fc2b_ref[0:1, :])


def _rows(i):
  return (i, 0)


def _fixed(*_):
  return (0, 0)


_B_X = pl.BlockSpec((_BM, 8), _rows)
_B_H = pl.BlockSpec((_BM, _H), _rows)
_B_D8 = pl.BlockSpec((_BM, 8), _rows)
_B_G = pl.BlockSpec((4, _BM, _QW), lambda i: (0, i, 0))
_B_A2 = pl.BlockSpec((2, _BM, _QW), lambda i: (0, i, 0))
_B_DEGP = pl.BlockSpec((2, _BM, _QW), lambda i: (0, i, 0))
_B_P8 = pl.BlockSpec((8, _H), _fixed)
_B_P832 = pl.BlockSpec((8, 32), _fixed)
_B_P88 = pl.BlockSpec((8, 8), _fixed)
_B_W = pl.BlockSpec((_H, _H), _fixed)
_B_W832 = pl.BlockSpec((8, _H), _fixed)
_B_FC1 = pl.BlockSpec((_H, 32), _fixed)
_B_FC2 = pl.BlockSpec((32, 8), _fixed)
_B_O = pl.BlockSpec((_BM, 8), _rows)

_enc = pl.pallas_call(
    _enc_body,
    grid=(_GRID,),
    in_specs=[_B_X, _B_DEGP, _B_W832, _B_P8, _B_P8, _B_P8, _B_W],
    out_specs=[_B_H, _B_D8, _B_G],
    out_shape=[
        jax.ShapeDtypeStruct((_N, _H), jnp.float32),
        jax.ShapeDtypeStruct((_N, 8), jnp.float32),
        jax.ShapeDtypeStruct((4, _N, _QW), jnp.float32),
    ],
)

_stat = pl.pallas_call(
    _stat_body,
    grid=(_GRID,),
    in_specs=[_B_A2, _B_A2, _B_G, _B_D8, _B_P8],
    out_specs=[_B_H, _B_P8],
    out_shape=[
        jax.ShapeDtypeStruct((_N, _H), jnp.float32),
        jax.ShapeDtypeStruct((8, _H), jnp.float32),
    ],
)

_apply = pl.pallas_call(
    _apply_body,
    grid=(_GRID,),
    in_specs=[_B_H, _B_P8, _B_P8, _B_P8, _B_H, _B_D8, _B_W],
    out_specs=[_B_H, _B_G],
    out_shape=[
        jax.ShapeDtypeStruct((_N, _H), jnp.float32),
        jax.ShapeDtypeStruct((4, _N, _QW), jnp.float32),
    ],
)

_final = pl.pallas_call(
    _final_body,
    grid=(_GRID,),
    in_specs=[_B_H, _B_P8, _B_P8, _B_P8, _B_H, _B_FC1, _B_P832, _B_FC2,
              _B_P88],
    out_specs=_B_O,
    out_shape=jax.ShapeDtypeStruct((_N, 8), jnp.float32),
)


def _row8(v):
  return jnp.broadcast_to(v.reshape(1, -1).astype(jnp.float32), (8, v.size))


def kernel(x, edge_index, W_enc, b_enc, ln_w, ln_b, bn_w, bn_b,
           W1, b1, W2, b2, W3, b3, fc1_w, fc1_b, fc2_w, fc2_b):
  src = edge_index[0].astype(jnp.int32)
  dst = edge_index[1].astype(jnp.int32)
  pad = _EP - _E
  srcp = jnp.concatenate([src, jnp.zeros((pad,), jnp.int32)])
  dstp = jnp.concatenate([dst, jnp.full((pad,), _N, jnp.int32)])
  srcq = jnp.stack([srcp, srcp + _N, srcp + 2 * _N, srcp + 3 * _N])
  srcq = srcq.reshape(2, 2, _ER, _LW)
  dst2 = dstp.reshape(_ER, _LW)

  xp = jnp.pad(x.astype(jnp.float32), ((0, 0), (0, 6)))
  wenc = jnp.pad(W_enc.astype(jnp.float32).T, ((0, 6), (0, 0)))
  w1t = W1.astype(jnp.float32).T
  w2t = W2.astype(jnp.float32).T
  w3t = W3.astype(jnp.float32).T
  fc1t = fc1_w.astype(jnp.float32).T
  fc2t = jnp.pad(fc2_w.astype(jnp.float32).T, ((0, 0), (0, 6)))
  fc2b = jnp.pad(fc2_b.astype(jnp.float32), (0, 6))

  degp = _sc_deg(dst2).reshape(2, _N, _QW)

  h0, dinv8, g = _enc(xp, degp, wenc, _row8(b_enc), _row8(ln_w),
                      _row8(ln_b), w1t)

  hid = h0
  for bvec, nxt in ((b1, w2t), (b2, w3t), (b3, None)):
    gt = g.reshape(4 * _N, _QW)
    agg_a = _sc_agg(gt, srcq[0], dst2).reshape(2, _N, _QW)
    agg_b = _sc_agg(gt, srcq[1], dst2).reshape(2, _N, _QW)
    t, st = _stat(agg_a, agg_b, g, dinv8, _row8(bvec))
    if nxt is not None:
      hid, g = _apply(t, st, _row8(bn_w), _row8(bn_b), hid, dinv8, nxt)
    else:
      out8 = _final(t, st, _row8(bn_w), _row8(bn_b), hid, fc1t,
                    _row8(fc1_b), fc2t, _row8(fc2b))
  return out8[:, :2]
